# Initial kernel scaffold; baseline (speedup 1.0000x reference)
import jax
import jax.numpy as jnp
from jax.experimental import pallas as pl


def kernel(x, partition_ids, W_emb1, b_emb1, W_emb2, b_emb2, W_g1, b_g1, W_g2, b_g2, W_go, b_go, W_c1, b_c1, ln1_g, ln1_b, W_c2, b_c2, ln2_g, ln2_b, W_c3, b_c3):
    raise NotImplementedError("write your pallas kernel here")


# speedup vs baseline: 19.9880x; 19.9880x over previous
import functools

import jax
import jax.numpy as jnp
from jax.experimental import pallas as pl

F32 = jnp.float32
BF16 = jnp.bfloat16


def _sanitize(v):
    v = jnp.where(jnp.isnan(v), 0.0, v)
    v = jnp.where(v == jnp.inf, 1.0, v)
    v = jnp.where(v == -jnp.inf, -1.0, v)
    return v


def _onehot_t(pid_row, pp, bn):
    iota = jax.lax.broadcasted_iota(jnp.int32, (pp, bn), 0)
    return (iota == pid_row).astype(F32)


def _pass_a(x_ref, pid_ref, b1_ref, bb1_ref, b2_ref, bb2_ref,
            meas_ref, s1_ref, *, pp, bn):
    i = pl.program_id(0)
    xb = _sanitize(x_ref[...])
    h = jnp.maximum(
        jnp.dot(xb, b1_ref[...], preferred_element_type=F32) + bb1_ref[...],
        0.0)
    meas = jnp.dot(h, b2_ref[...], preferred_element_type=F32) + bb2_ref[...]
    meas_ref[...] = meas
    oht = _onehot_t(pid_ref[0], pp, bn)
    contrib = jnp.dot(oht, meas, preferred_element_type=F32)

    @pl.when(i == 0)
    def _():
        s1_ref[...] = jnp.zeros_like(s1_ref)

    s1_ref[...] += contrib


def _mid1(s1_ref, agg1_ref):
    s1 = s1_ref[...]
    cnt = jnp.maximum(s1[:, 24:25], 1.0)
    agg1_ref[...] = s1 / cnt


def _pass_b(meas_ref, pid_ref, agg1_ref, bg1_ref, bbg1_ref,
            feats1_ref, s2_ref, *, pp, bn):
    i = pl.program_id(0)
    oht = _onehot_t(pid_ref[0], pp, bn)
    gath = jax.lax.dot_general(oht, agg1_ref[...],
                               (((0,), (0,)), ((), ())),
                               preferred_element_type=F32)
    feats1 = meas_ref[...] + gath
    feats1_ref[...] = feats1
    h1 = jnp.maximum(
        jnp.dot(feats1, bg1_ref[...], preferred_element_type=F32)
        + bbg1_ref[...], 0.0)
    contrib = jnp.dot(oht, h1, preferred_element_type=F32)

    @pl.when(i == 0)
    def _():
        s2_ref[...] = jnp.zeros_like(s2_ref)

    s2_ref[...] += contrib


def _mid2(s2_ref, s1_ref, bg2_ref, ag2_ref):
    cnt = jnp.maximum(s1_ref[:, 24:25], 1.0)
    agg2 = s2_ref[...] / cnt
    ag2_ref[...] = jnp.dot(agg2, bg2_ref[...], preferred_element_type=F32)


def _layer_norm(h, g, b):
    mu = jnp.mean(h, axis=-1, keepdims=True)
    var = jnp.mean((h - mu) * (h - mu), axis=-1, keepdims=True)
    return (h - mu) / jnp.sqrt(var + 1e-5) * g + b


def _pass_c(feats1_ref, pid_ref, ag2_ref, bg1_ref, bbg1_ref, bg2_ref,
            bbg2_ref, gw1_ref, c1b_ref, ln1g_ref, ln1b_ref, wc2_ref, bc2_ref,
            ln2g_ref, ln2b_ref, wc3_ref, bc3_ref, out_ref, *, pp, bn):
    h1 = jnp.maximum(
        jnp.dot(feats1_ref[...], bg1_ref[...], preferred_element_type=F32)
        + bbg1_ref[...], 0.0)
    z = jnp.dot(h1, bg2_ref[...], preferred_element_type=F32) + bbg2_ref[...]
    oht = _onehot_t(pid_ref[0], pp, bn)
    gath2 = jax.lax.dot_general(oht, ag2_ref[...],
                                (((0,), (0,)), ((), ())),
                                preferred_element_type=F32)
    h2 = jnp.maximum(z + gath2, 0.0)
    c = jnp.dot(h2, gw1_ref[...], preferred_element_type=F32) + c1b_ref[...]
    c = _layer_norm(c, ln1g_ref[...], ln1b_ref[...])
    c = jnp.maximum(c, 0.0)
    c = jnp.dot(c, wc2_ref[...], preferred_element_type=F32) + bc2_ref[...]
    c = _layer_norm(c, ln2g_ref[...], ln2b_ref[...])
    c = jnp.maximum(c, 0.0)
    out_ref[...] = jnp.dot(c, wc3_ref[...],
                           preferred_element_type=F32) + bc3_ref[...]


def kernel(x, partition_ids, W_emb1, b_emb1, W_emb2, b_emb2, W_g1, b_g1,
           W_g2, b_g2, W_go, b_go, W_c1, b_c1, ln1_g, ln1_b, W_c2, b_c2,
           ln2_g, ln2_b, W_c3, b_c3):
    n, a, d_in = x.shape
    h_dim = W_emb1.shape[1]
    m = W_emb2.shape[1]
    p = 1000
    pp = 1024
    ad = a * d_in
    ah = a * h_dim
    am = a * m
    amp = 32

    bn = 2000
    for cand in (2000, 1000, 800, 500, 200, 100, 50, 40, 25, 20, 10, 8):
        if n % cand == 0:
            bn = cand
            break
    nb = n // bn

    eye_a = jnp.eye(a, dtype=F32)
    b1 = jnp.kron(eye_a, W_emb1)
    bb1 = jnp.tile(b_emb1, a)[None, :]
    b2 = jnp.kron(eye_a, W_emb2)
    b2 = jnp.pad(b2, ((0, 0), (0, amp - am)))
    bb2 = jnp.pad(jnp.tile(b_emb2, a), (0, amp - am))
    bb2 = bb2.at[am].set(1.0)[None, :]
    bg1 = jnp.pad(jnp.kron(eye_a, W_g1), ((0, amp - am), (0, 0)))
    bbg1 = jnp.tile(b_g1, a)[None, :]
    bg2 = jnp.kron(eye_a, W_g2)
    bbg2 = jnp.tile(b_g2, a)[None, :]
    bgo = jnp.kron(eye_a, W_go)
    pool_t = jnp.kron(jnp.full((a, 1), 1.0 / a, dtype=F32),
                      jnp.eye(m, dtype=F32))
    g_mat = bgo @ pool_t
    g0 = b_go
    gw1 = g_mat @ W_c1
    c1b = (g0 @ W_c1 + b_c1)[None, :]

    x2 = x.reshape(n, ad)
    pid3 = partition_ids.astype(jnp.int32).reshape(nb, 1, bn)

    full = lambda shp: pl.BlockSpec(shp, lambda i: tuple(0 for _ in shp))
    row_block = lambda shp: pl.BlockSpec(shp, lambda i: (i,) + (0,) * (len(shp) - 1))

    meas, s1 = pl.pallas_call(
        functools.partial(_pass_a, pp=pp, bn=bn),
        grid=(nb,),
        in_specs=[
            row_block((bn, ad)),
            row_block((1, 1, bn)),
            full((ad, ah)),
            full((1, ah)),
            full((ah, amp)),
            full((1, amp)),
        ],
        out_specs=[
            row_block((bn, amp)),
            full((pp, amp)),
        ],
        out_shape=[
            jax.ShapeDtypeStruct((n, amp), F32),
            jax.ShapeDtypeStruct((pp, amp), F32),
        ],
    )(x2, pid3, b1, bb1, b2, bb2)

    agg1 = pl.pallas_call(
        _mid1,
        out_shape=jax.ShapeDtypeStruct((pp, amp), F32),
    )(s1)

    feats1, s2 = pl.pallas_call(
        functools.partial(_pass_b, pp=pp, bn=bn),
        grid=(nb,),
        in_specs=[
            row_block((bn, amp)),
            row_block((1, 1, bn)),
            full((pp, amp)),
            full((amp, ah)),
            full((1, ah)),
        ],
        out_specs=[
            row_block((bn, amp)),
            full((pp, ah)),
        ],
        out_shape=[
            jax.ShapeDtypeStruct((n, amp), F32),
            jax.ShapeDtypeStruct((pp, ah), F32),
        ],
    )(meas, pid3, agg1, bg1, bbg1)

    ag2 = pl.pallas_call(
        _mid2,
        out_shape=jax.ShapeDtypeStruct((pp, ah), F32),
    )(s2, s1, bg2)

    scores = pl.pallas_call(
        functools.partial(_pass_c, pp=pp, bn=bn),
        grid=(nb,),
        in_specs=[
            row_block((bn, amp)),
            row_block((1, 1, bn)),
            full((pp, ah)),
            full((amp, ah)),
            full((1, ah)),
            full((ah, ah)),
            full((1, ah)),
            full((ah, h_dim)),
            full((1, h_dim)),
            full((1, h_dim)),
            full((1, h_dim)),
            full((h_dim, h_dim // 2)),
            full((1, h_dim // 2)),
            full((1, h_dim // 2)),
            full((1, h_dim // 2)),
            full((h_dim // 2, 1)),
            full((1, 1)),
        ],
        out_specs=row_block((bn, 1)),
        out_shape=jax.ShapeDtypeStruct((n, 1), F32),
    )(feats1, pid3, ag2, bg1, bbg1, bg2, bbg2, gw1, c1b,
      ln1_g[None, :], ln1_b[None, :], W_c2, b_c2[None, :],
      ln2_g[None, :], ln2_b[None, :], W_c3, b_c3[None, :])

    return scores[:, 0]


# speedup vs baseline: 20.3632x; 1.0188x over previous
import functools

import jax
import jax.numpy as jnp
from jax import lax
from jax.experimental import pallas as pl
from jax.experimental.pallas import tpu as pltpu
from jax.experimental.pallas import tpu_sc as plsc

F32 = jnp.float32

NP_PAD = 102400
N_WORKERS = 32
ROWS_PER_W = NP_PAD // N_WORKERS
CH = 128
NCH = ROWS_PER_W // CH


def _sanitize(v):
    v = jnp.where(jnp.isnan(v), 0.0, v)
    v = jnp.where(v == jnp.inf, 1.0, v)
    v = jnp.where(v == -jnp.inf, -1.0, v)
    return v


def _onehot_t(pid_row, pp, bn):
    iota = jax.lax.broadcasted_iota(jnp.int32, (pp, bn), 0)
    return (iota == pid_row).astype(F32)


def _pass_a(x_ref, pid_ref, b1_ref, bb1_ref, b2_ref, bb2_ref, meas_ref,
            s1_ref, *, pp, bn):
    i = pl.program_id(0)
    xb = _sanitize(x_ref[...])
    h = jnp.maximum(
        jnp.dot(xb, b1_ref[...], preferred_element_type=F32) + bb1_ref[...],
        0.0)
    meas = jnp.dot(h, b2_ref[...], preferred_element_type=F32) + bb2_ref[...]
    meas_ref[...] = meas
    oht = _onehot_t(pid_ref[0], pp, bn)
    contrib = jnp.dot(oht, meas, preferred_element_type=F32)

    @pl.when(i == 0)
    def _():
        s1_ref[...] = jnp.zeros_like(s1_ref)

    s1_ref[...] += contrib


def _sc_scatter_body(meas_hbm, pid_hbm, zero_hbm, out_hbm, idx_v, rows_v,
                     acc_sh):
    c = lax.axis_index("c")
    s = lax.axis_index("s")
    wid = s * 2 + c
    base = wid * ROWS_PER_W

    @pl.when(s == 0)
    def _():
        pltpu.sync_copy(zero_hbm, acc_sh)

    plsc.subcore_barrier()
    for j in range(NCH):
        off = base + j * CH
        pltpu.sync_copy(pid_hbm.at[pl.ds(off, CH)], idx_v)
        pltpu.sync_copy(meas_hbm.at[pl.ds(off, CH)], rows_v)
        pltpu.sync_copy(rows_v, acc_sh.at[idx_v], add=True)
    plsc.subcore_barrier()

    @pl.when(s == 0)
    def _():
        pltpu.sync_copy(acc_sh, out_hbm.at[c])


def _sc_scatter(meas_p, pid_p, zeros):
    mesh = plsc.VectorSubcoreMesh(core_axis_name="c", subcore_axis_name="s")
    return pl.kernel(
        _sc_scatter_body,
        mesh=mesh,
        out_type=jax.ShapeDtypeStruct((2, 1024, 32), F32),
        scratch_types=[
            pltpu.VMEM((CH,), jnp.int32),
            pltpu.VMEM((CH, 32), F32),
            pltpu.VMEM_SHARED((1024, 32), F32),
        ],
    )(meas_p, pid_p, zeros)


def _pass_b(meas_ref, pid_ref, s1_ref, bg1_ref, bbg1_ref,
            feats1_ref, s2_ref, agg1_scr, *, pp, bn):
    i = pl.program_id(0)

    @pl.when(i == 0)
    def _():
        s1 = s1_ref[...]
        cnt = jnp.maximum(s1[:, 24:25], 1.0)
        agg1_scr[...] = s1 / cnt
        s2_ref[...] = jnp.zeros_like(s2_ref)

    oht = _onehot_t(pid_ref[0], pp, bn)
    gath = jax.lax.dot_general(oht, agg1_scr[...],
                               (((0,), (0,)), ((), ())),
                               preferred_element_type=F32)
    feats1 = meas_ref[...] + gath
    feats1_ref[...] = feats1
    h1 = jnp.maximum(
        jnp.dot(feats1, bg1_ref[...], preferred_element_type=F32)
        + bbg1_ref[...], 0.0)
    s2_ref[...] += jnp.dot(oht, h1, preferred_element_type=F32)


def _layer_norm(h, g, b):
    mu = jnp.mean(h, axis=-1, keepdims=True)
    var = jnp.mean((h - mu) * (h - mu), axis=-1, keepdims=True)
    return (h - mu) / jnp.sqrt(var + 1e-5) * g + b


def _pass_c(feats1_ref, pid_ref, s2_ref, s1_ref, bg1_ref, bbg1_ref, bg2_ref,
            bbg2_ref, gw1_ref, c1b_ref, ln1g_ref, ln1b_ref, wc2_ref, bc2_ref,
            ln2g_ref, ln2b_ref, wc3_ref, bc3_ref, out_ref, ag2_scr, *, pp, bn):
    i = pl.program_id(0)

    @pl.when(i == 0)
    def _():
        s1 = s1_ref[...]
        cnt = jnp.maximum(s1[:, 24:25], 1.0)
        agg2 = s2_ref[...] / cnt
        ag2_scr[...] = jnp.dot(agg2, bg2_ref[...], preferred_element_type=F32)

    h1 = jnp.maximum(
        jnp.dot(feats1_ref[...], bg1_ref[...], preferred_element_type=F32)
        + bbg1_ref[...], 0.0)
    z = jnp.dot(h1, bg2_ref[...], preferred_element_type=F32) + bbg2_ref[...]
    oht = _onehot_t(pid_ref[0], pp, bn)
    gath2 = jax.lax.dot_general(oht, ag2_scr[...],
                                (((0,), (0,)), ((), ())),
                                preferred_element_type=F32)
    h2 = jnp.maximum(z + gath2, 0.0)
    c = jnp.dot(h2, gw1_ref[...], preferred_element_type=F32) + c1b_ref[...]
    c = _layer_norm(c, ln1g_ref[...], ln1b_ref[...])
    c = jnp.maximum(c, 0.0)
    c = jnp.dot(c, wc2_ref[...], preferred_element_type=F32) + bc2_ref[...]
    c = _layer_norm(c, ln2g_ref[...], ln2b_ref[...])
    c = jnp.maximum(c, 0.0)
    out_ref[...] = jnp.dot(c, wc3_ref[...],
                           preferred_element_type=F32) + bc3_ref[...]


def kernel(x, partition_ids, W_emb1, b_emb1, W_emb2, b_emb2, W_g1, b_g1,
           W_g2, b_g2, W_go, b_go, W_c1, b_c1, ln1_g, ln1_b, W_c2, b_c2,
           ln2_g, ln2_b, W_c3, b_c3):
    n, a, d_in = x.shape
    h_dim = W_emb1.shape[1]
    m = W_emb2.shape[1]
    pp = 1024
    ad = a * d_in
    ah = a * h_dim
    am = a * m
    amp = 32

    bn = 2000
    for cand in (2000, 1000, 800, 500, 200, 100, 50, 40, 25, 20, 10, 8):
        if n % cand == 0:
            bn = cand
            break
    nb = n // bn

    eye_a = jnp.eye(a, dtype=F32)
    b1 = jnp.kron(eye_a, W_emb1)
    bb1 = jnp.tile(b_emb1, a)[None, :]
    b2 = jnp.kron(eye_a, W_emb2)
    b2 = jnp.pad(b2, ((0, 0), (0, amp - am)))
    bb2 = jnp.pad(jnp.tile(b_emb2, a), (0, amp - am))
    bb2 = bb2.at[am].set(1.0)[None, :]
    bg1 = jnp.pad(jnp.kron(eye_a, W_g1), ((0, amp - am), (0, 0)))
    bbg1 = jnp.tile(b_g1, a)[None, :]
    bg2 = jnp.kron(eye_a, W_g2)
    bbg2 = jnp.tile(b_g2, a)[None, :]
    bgo = jnp.kron(eye_a, W_go)
    pool_t = jnp.kron(jnp.full((a, 1), 1.0 / a, dtype=F32),
                      jnp.eye(m, dtype=F32))
    g_mat = bgo @ pool_t
    gw1 = g_mat @ W_c1
    c1b = (b_go @ W_c1 + b_c1)[None, :]

    x2 = x.reshape(n, ad)
    pid32 = partition_ids.astype(jnp.int32)
    pid3 = pid32.reshape(nb, 1, bn)

    full = lambda shp: pl.BlockSpec(shp, lambda i: tuple(0 for _ in shp))
    row_block = lambda shp: pl.BlockSpec(shp, lambda i: (i,) + (0,) * (len(shp) - 1))

    meas, s1 = pl.pallas_call(
        functools.partial(_pass_a, pp=pp, bn=bn),
        grid=(nb,),
        in_specs=[
            row_block((bn, ad)),
            row_block((1, 1, bn)),
            full((ad, ah)),
            full((1, ah)),
            full((ah, amp)),
            full((1, amp)),
        ],
        out_specs=[
            row_block((bn, amp)),
            full((pp, amp)),
        ],
        out_shape=[
            jax.ShapeDtypeStruct((n, amp), F32),
            jax.ShapeDtypeStruct((pp, amp), F32),
        ],
    )(x2, pid3, b1, bb1, b2, bb2)

    feats1, s2 = pl.pallas_call(
        functools.partial(_pass_b, pp=pp, bn=bn),
        grid=(nb,),
        in_specs=[
            row_block((bn, amp)),
            row_block((1, 1, bn)),
            full((pp, amp)),
            full((amp, ah)),
            full((1, ah)),
        ],
        out_specs=[
            row_block((bn, amp)),
            full((pp, ah)),
        ],
        out_shape=[
            jax.ShapeDtypeStruct((n, amp), F32),
            jax.ShapeDtypeStruct((pp, ah), F32),
        ],
        scratch_shapes=[pltpu.VMEM((pp, amp), F32)],
    )(meas, pid3, s1, bg1, bbg1)

    scores = pl.pallas_call(
        functools.partial(_pass_c, pp=pp, bn=bn),
        grid=(nb,),
        in_specs=[
            row_block((bn, amp)),
            row_block((1, 1, bn)),
            full((pp, ah)),
            full((pp, amp)),
            full((amp, ah)),
            full((1, ah)),
            full((ah, ah)),
            full((1, ah)),
            full((ah, h_dim)),
            full((1, h_dim)),
            full((1, h_dim)),
            full((1, h_dim)),
            full((h_dim, h_dim // 2)),
            full((1, h_dim // 2)),
            full((1, h_dim // 2)),
            full((1, h_dim // 2)),
            full((h_dim // 2, 1)),
            full((1, 1)),
        ],
        out_specs=row_block((bn, 1)),
        out_shape=jax.ShapeDtypeStruct((n, 1), F32),
        scratch_shapes=[pltpu.VMEM((pp, ah), F32)],
    )(feats1, pid3, s2, s1, bg1, bbg1, bg2, bbg2, gw1, c1b,
      ln1_g[None, :], ln1_b[None, :], W_c2, b_c2[None, :],
      ln2_g[None, :], ln2_b[None, :], W_c3, b_c3[None, :])

    return scores[:, 0]


# speedup vs baseline: 20.9290x; 1.0278x over previous
import functools

import jax
import jax.numpy as jnp
from jax import lax
from jax.experimental import pallas as pl
from jax.experimental.pallas import tpu as pltpu
from jax.experimental.pallas import tpu_sc as plsc

F32 = jnp.float32

NP_PAD = 102400
N_WORKERS = 32
ROWS_PER_W = NP_PAD // N_WORKERS
CH = 128
NCH = ROWS_PER_W // CH


def _sanitize(v):
    v = jnp.where(jnp.isnan(v), 0.0, v)
    v = jnp.where(v == jnp.inf, 1.0, v)
    v = jnp.where(v == -jnp.inf, -1.0, v)
    return v


def _onehot_t(pid_row, pp, bn):
    iota = jax.lax.broadcasted_iota(jnp.int32, (pp, bn), 0)
    return (iota == pid_row).astype(F32)


def _pass_a(x_ref, pid_ref, b1_ref, bb1_ref, b2_ref, bb2_ref, meas_ref,
            s1_ref, *, pp, bn):
    i = pl.program_id(0)
    xb = _sanitize(x_ref[...])
    h = jnp.maximum(
        jnp.dot(xb, b1_ref[...], preferred_element_type=F32) + bb1_ref[...],
        0.0)
    meas = jnp.dot(h, b2_ref[...], preferred_element_type=F32) + bb2_ref[...]
    meas_ref[...] = meas
    oht = _onehot_t(pid_ref[0], pp, bn)
    contrib = jnp.dot(oht, meas, preferred_element_type=F32)

    @pl.when(i == 0)
    def _():
        s1_ref[...] = jnp.zeros_like(s1_ref)

    s1_ref[...] += contrib


def _sc_scatter_body(meas_hbm, pid_hbm, zero_hbm, out_hbm, idx_v, rows_v,
                     acc_sh):
    c = lax.axis_index("c")
    s = lax.axis_index("s")
    wid = s * 2 + c
    base = wid * ROWS_PER_W

    @pl.when(s == 0)
    def _():
        pltpu.sync_copy(zero_hbm, acc_sh)

    plsc.subcore_barrier()
    for j in range(NCH):
        off = base + j * CH
        pltpu.sync_copy(pid_hbm.at[pl.ds(off, CH)], idx_v)
        pltpu.sync_copy(meas_hbm.at[pl.ds(off, CH)], rows_v)
        pltpu.sync_copy(rows_v, acc_sh.at[idx_v], add=True)
    plsc.subcore_barrier()

    @pl.when(s == 0)
    def _():
        pltpu.sync_copy(acc_sh, out_hbm.at[c])


def _sc_scatter(meas_p, pid_p, zeros):
    mesh = plsc.VectorSubcoreMesh(core_axis_name="c", subcore_axis_name="s")
    return pl.kernel(
        _sc_scatter_body,
        mesh=mesh,
        out_type=jax.ShapeDtypeStruct((2, 1024, 32), F32),
        scratch_types=[
            pltpu.VMEM((CH,), jnp.int32),
            pltpu.VMEM((CH, 32), F32),
            pltpu.VMEM_SHARED((1024, 32), F32),
        ],
    )(meas_p, pid_p, zeros)


def _pass_b(meas_ref, pid_ref, s1_ref, bg1_ref, bbg1_ref,
            feats1_ref, s2_ref, agg1_scr, *, pp, bn):
    i = pl.program_id(0)

    @pl.when(i == 0)
    def _():
        s1 = s1_ref[...]
        cnt = jnp.maximum(s1[:, 24:25], 1.0)
        agg1_scr[...] = s1 / cnt
        s2_ref[...] = jnp.zeros_like(s2_ref)

    oht = _onehot_t(pid_ref[0], pp, bn)
    gath = jax.lax.dot_general(oht, agg1_scr[...],
                               (((0,), (0,)), ((), ())),
                               preferred_element_type=F32)
    feats1 = meas_ref[...] + gath
    feats1_ref[...] = feats1
    h1 = jnp.maximum(
        jnp.dot(feats1, bg1_ref[...], preferred_element_type=F32)
        + bbg1_ref[...], 0.0)
    s2_ref[...] += jnp.dot(oht, h1, preferred_element_type=F32)


def _layer_norm(h, g, b):
    mu = jnp.mean(h, axis=-1, keepdims=True)
    var = jnp.mean((h - mu) * (h - mu), axis=-1, keepdims=True)
    return (h - mu) / jnp.sqrt(var + 1e-5) * g + b


def _pass_c(feats1_ref, pid_ref, s2_ref, s1_ref, bg1_ref, bbg1_ref, bg2_ref,
            bbg2_ref, gw1_ref, c1b_ref, ln1g_ref, ln1b_ref, wc2_ref, bc2_ref,
            ln2g_ref, ln2b_ref, wc3_ref, bc3_ref, out_ref, ag2_scr, *, pp, bn):
    i = pl.program_id(0)

    @pl.when(i == 0)
    def _():
        s1 = s1_ref[...]
        cnt = jnp.maximum(s1[:, 24:25], 1.0)
        agg2 = s2_ref[...] / cnt
        ag2_scr[...] = jnp.dot(agg2, bg2_ref[...], preferred_element_type=F32)

    h1 = jnp.maximum(
        jnp.dot(feats1_ref[...], bg1_ref[...], preferred_element_type=F32)
        + bbg1_ref[...], 0.0)
    z = jnp.dot(h1, bg2_ref[...], preferred_element_type=F32) + bbg2_ref[...]
    oht = _onehot_t(pid_ref[0], pp, bn)
    gath2 = jax.lax.dot_general(oht, ag2_scr[...],
                                (((0,), (0,)), ((), ())),
                                preferred_element_type=F32)
    h2 = jnp.maximum(z + gath2, 0.0)
    c = jnp.dot(h2, gw1_ref[...], preferred_element_type=F32) + c1b_ref[...]
    c = _layer_norm(c, ln1g_ref[...], ln1b_ref[...])
    c = jnp.maximum(c, 0.0)
    c = jnp.dot(c, wc2_ref[...], preferred_element_type=F32) + bc2_ref[...]
    c = _layer_norm(c, ln2g_ref[...], ln2b_ref[...])
    c = jnp.maximum(c, 0.0)
    out_ref[...] = jnp.dot(c, wc3_ref[...],
                           preferred_element_type=F32) + bc3_ref[...]


def kernel(x, partition_ids, W_emb1, b_emb1, W_emb2, b_emb2, W_g1, b_g1,
           W_g2, b_g2, W_go, b_go, W_c1, b_c1, ln1_g, ln1_b, W_c2, b_c2,
           ln2_g, ln2_b, W_c3, b_c3):
    n, a, d_in = x.shape
    h_dim = W_emb1.shape[1]
    m = W_emb2.shape[1]
    pp = 1024
    ad = a * d_in
    ah = a * h_dim
    am = a * m
    amp = 32

    bn = 2000
    for cand in (4000, 2000, 1000, 800, 500, 200, 100, 50, 40, 25, 20, 10, 8):
        if n % cand == 0:
            bn = cand
            break
    nb = n // bn

    eye_a = jnp.eye(a, dtype=F32)
    b1 = jnp.kron(eye_a, W_emb1)
    bb1 = jnp.tile(b_emb1, a)[None, :]
    b2 = jnp.kron(eye_a, W_emb2)
    b2 = jnp.pad(b2, ((0, 0), (0, amp - am)))
    bb2 = jnp.pad(jnp.tile(b_emb2, a), (0, amp - am))
    bb2 = bb2.at[am].set(1.0)[None, :]
    bg1 = jnp.pad(jnp.kron(eye_a, W_g1), ((0, amp - am), (0, 0)))
    bbg1 = jnp.tile(b_g1, a)[None, :]
    bg2 = jnp.kron(eye_a, W_g2)
    bbg2 = jnp.tile(b_g2, a)[None, :]
    bgo = jnp.kron(eye_a, W_go)
    pool_t = jnp.kron(jnp.full((a, 1), 1.0 / a, dtype=F32),
                      jnp.eye(m, dtype=F32))
    g_mat = bgo @ pool_t
    gw1 = g_mat @ W_c1
    c1b = (b_go @ W_c1 + b_c1)[None, :]

    x2 = x.reshape(n, ad)
    pid32 = partition_ids.astype(jnp.int32)
    pid3 = pid32.reshape(nb, 1, bn)

    full = lambda shp: pl.BlockSpec(shp, lambda i: tuple(0 for _ in shp))
    row_block = lambda shp: pl.BlockSpec(shp, lambda i: (i,) + (0,) * (len(shp) - 1))

    meas, s1 = pl.pallas_call(
        functools.partial(_pass_a, pp=pp, bn=bn),
        grid=(nb,),
        in_specs=[
            row_block((bn, ad)),
            row_block((1, 1, bn)),
            full((ad, ah)),
            full((1, ah)),
            full((ah, amp)),
            full((1, amp)),
        ],
        out_specs=[
            row_block((bn, amp)),
            full((pp, amp)),
        ],
        out_shape=[
            jax.ShapeDtypeStruct((n, amp), F32),
            jax.ShapeDtypeStruct((pp, amp), F32),
        ],
    )(x2, pid3, b1, bb1, b2, bb2)

    feats1, s2 = pl.pallas_call(
        functools.partial(_pass_b, pp=pp, bn=bn),
        grid=(nb,),
        in_specs=[
            row_block((bn, amp)),
            row_block((1, 1, bn)),
            full((pp, amp)),
            full((amp, ah)),
            full((1, ah)),
        ],
        out_specs=[
            row_block((bn, amp)),
            full((pp, ah)),
        ],
        out_shape=[
            jax.ShapeDtypeStruct((n, amp), F32),
            jax.ShapeDtypeStruct((pp, ah), F32),
        ],
        scratch_shapes=[pltpu.VMEM((pp, amp), F32)],
    )(meas, pid3, s1, bg1, bbg1)

    scores = pl.pallas_call(
        functools.partial(_pass_c, pp=pp, bn=bn),
        grid=(nb,),
        in_specs=[
            row_block((bn, amp)),
            row_block((1, 1, bn)),
            full((pp, ah)),
            full((pp, amp)),
            full((amp, ah)),
            full((1, ah)),
            full((ah, ah)),
            full((1, ah)),
            full((ah, h_dim)),
            full((1, h_dim)),
            full((1, h_dim)),
            full((1, h_dim)),
            full((h_dim, h_dim // 2)),
            full((1, h_dim // 2)),
            full((1, h_dim // 2)),
            full((1, h_dim // 2)),
            full((h_dim // 2, 1)),
            full((1, 1)),
        ],
        out_specs=row_block((bn, 1)),
        out_shape=jax.ShapeDtypeStruct((n, 1), F32),
        scratch_shapes=[pltpu.VMEM((pp, ah), F32)],
    )(feats1, pid3, s2, s1, bg1, bbg1, bg2, bbg2, gw1, c1b,
      ln1_g[None, :], ln1_b[None, :], W_c2, b_c2[None, :],
      ln2_g[None, :], ln2_b[None, :], W_c3, b_c3[None, :])

    return scores[:, 0]
